# Initial kernel scaffold; baseline (speedup 1.0000x reference)
import jax
import jax.numpy as jnp
from jax.experimental import pallas as pl


def kernel(x, emb_table, W, b):
    raise NotImplementedError("write your pallas kernel here")


# speedup vs baseline: 5.8662x; 5.8662x over previous
import functools

import jax
import jax.numpy as jnp
from jax import lax
from jax.experimental import pallas as pl
from jax.experimental.pallas import tpu as pltpu
from jax.experimental.pallas import tpu_sc as plsc

NC = 2
NS = 16
L = 16
IDX_BLK = 128


def _tc_table_matvec(emb_table, W):
    V, E = emb_table.shape
    BLK = 8000

    def body(tbl_ref, w_ref, out_ref):
        out_ref[...] = lax.dot_general(
            tbl_ref[...], w_ref[...],
            dimension_numbers=(((1,), (1,)), ((), ())),
            preferred_element_type=jnp.float32,
        )

    out = pl.pallas_call(
        body,
        grid=(V // BLK,),
        in_specs=[
            pl.BlockSpec((BLK, E), lambda i: (i, 0)),
            pl.BlockSpec((1, E), lambda i: (0, 0)),
        ],
        out_specs=pl.BlockSpec((BLK, 1), lambda i: (i, 0)),
        out_shape=jax.ShapeDtypeStruct((V, 1), jnp.float32),
    )(emb_table, W)
    return out.reshape(V)


def _sc_pool_head(x_perm, tw, bv, B, T):
    NW = NC * NS
    RPW = B // NW
    CB = L
    CHUNK_IDX = CB * T
    NBLK = CHUNK_IDX // IDX_BLK
    NCHUNK = RPW // CB

    mesh = plsc.VectorSubcoreMesh(core_axis_name="c", subcore_axis_name="s")

    @functools.partial(
        pl.kernel,
        out_type=jax.ShapeDtypeStruct((B,), jnp.float32),
        mesh=mesh,
        scratch_types=[
            pltpu.VMEM((CHUNK_IDX,), jnp.int32),
            pltpu.VMEM((CHUNK_IDX,), jnp.float32),
            pltpu.VMEM((L,), jnp.float32),
            pltpu.VMEM((RPW,), jnp.float32),
            pltpu.SemaphoreType.DMA,
        ],
    )
    def sc_kernel(x_hbm, tw_hbm, b_hbm, out_hbm,
                  idx_v, val_v, b_v, out_v, sem):
        wid = lax.axis_index("s") * NC + lax.axis_index("c")
        pltpu.sync_copy(b_hbm, b_v)
        bvec = b_v[...]
        xoff0 = wid * (RPW * T)

        def chunk_body(g, carry):
            pltpu.sync_copy(x_hbm.at[pl.ds(xoff0 + g * CHUNK_IDX, CHUNK_IDX)],
                            idx_v)
            handles = [
                pltpu.async_copy(
                    tw_hbm.at[idx_v.at[pl.ds(j * IDX_BLK, IDX_BLK)]],
                    val_v.at[pl.ds(j * IDX_BLK, IDX_BLK)],
                    sem,
                )
                for j in range(NBLK)
            ]
            for h in handles:
                h.wait()

            def acc_body(i, accs):
                a0, a1, a2, a3 = accs
                t = i * (8 * L)
                a0 = a0 + val_v[pl.ds(t + 0 * L, L)] + val_v[pl.ds(t + 4 * L, L)]
                a1 = a1 + val_v[pl.ds(t + 1 * L, L)] + val_v[pl.ds(t + 5 * L, L)]
                a2 = a2 + val_v[pl.ds(t + 2 * L, L)] + val_v[pl.ds(t + 6 * L, L)]
                a3 = a3 + val_v[pl.ds(t + 3 * L, L)] + val_v[pl.ds(t + 7 * L, L)]
                return (a0, a1, a2, a3)

            z = jnp.zeros((L,), jnp.float32)
            a0, a1, a2, a3 = lax.fori_loop(0, T // 8, acc_body, (z, z, z, z))
            res = (a0 + a1) + (a2 + a3)
            y = res * (1.0 / T) + bvec
            sgm = 1.0 / (1.0 + jnp.exp(-y))
            scaled = sgm * 10000.0
            rnd = (scaled + 8388608.0) - 8388608.0
            out_v[pl.ds(g * CB, CB)] = rnd / 10000.0
            return carry

        lax.fori_loop(0, NCHUNK, chunk_body, 0)
        pltpu.sync_copy(out_v, out_hbm.at[pl.ds(wid * RPW, RPW)])

    return sc_kernel(x_perm, tw, bv)


def kernel(x, emb_table, W, b):
    B, T = x.shape
    tw = _tc_table_matvec(emb_table.astype(jnp.float32), W.astype(jnp.float32))
    x_perm = (x.astype(jnp.int32)
              .reshape(B // L, L, T)
              .transpose(0, 2, 1)
              .reshape(B * T))
    bv = jnp.broadcast_to(b.reshape(1), (L,)).astype(jnp.float32)
    out = _sc_pool_head(x_perm, tw, bv, B, T)
    return out.reshape(B, 1)
